# baseline (speedup 1.0000x reference)
import functools

import jax
import jax.numpy as jnp
from jax import lax
from jax.experimental import pallas as pl
from jax.experimental.pallas import tpu as pltpu
from jax.experimental.pallas import tpu_sc as plsc

VOCAB = 5000
EMBED = 128
CLASSES = 20
P = 2
B = 1024
L = 50

NC = 2
NS = 16
NWORK = NC * NS

N_IDS = B * L
N_EDGE = B * L * 2 * P

IDS_PW = N_IDS // NWORK
EDGE_PW = N_EDGE // NWORK
ROWS_PW = B // NWORK

EDGE_CH = 128
N_EDGE_CH = EDGE_PW // EDGE_CH
EMB_CH = 80
N_EMB_CH = IDS_PW // EMB_CH
NW_CH = 64
N_NW_CH = IDS_PW // NW_CH
EDGE_MAIN = 24999936
EDGE_MAX = VOCAB * VOCAB - 1


def _sc_emb_kernel(emb_hbm, nodew_hbm, xidx_hbm,
                   g_hbm, nwv_hbm,
                   xidx_v, rows0, rows1, nwv_v,
                   semg0, semg1, semw0, semw1, semn):
    wid = lax.axis_index("s") * NC + lax.axis_index("c")
    r0 = wid * IDS_PW

    pltpu.sync_copy(xidx_hbm.at[pl.ds(r0, IDS_PW)], xidx_v)

    def nw_fire(c, carry):
        idx = xidx_v.at[pl.ds(c * NW_CH, NW_CH)]
        pltpu.async_copy(nodew_hbm.at[idx], nwv_v.at[pl.ds(c * NW_CH, NW_CH)],
                         semn)
        return carry

    lax.fori_loop(0, N_NW_CH, nw_fire, 0)

    def gather_chunk(c, buf, sem):
        idx = xidx_v.at[pl.ds(c * EMB_CH, EMB_CH)]
        pltpu.async_copy(emb_hbm.at[idx], buf, sem)

    def write_chunk(c, buf, sem):
        return pltpu.async_copy(buf, g_hbm.at[pl.ds(r0 + c * EMB_CH, EMB_CH)],
                                sem)

    def wait_gather(buf, sem):
        pltpu.make_async_copy(emb_hbm.at[pl.ds(0, EMB_CH), :], buf, sem).wait()

    def wait_write(c, buf, sem):
        pltpu.make_async_copy(buf, g_hbm.at[pl.ds(r0 + c * EMB_CH, EMB_CH)],
                              sem).wait()

    gather_chunk(0, rows0, semg0)

    def emb_body(r, carry):
        i = 2 * r
        gather_chunk(i + 1, rows1, semg1)
        wait_gather(rows0, semg0)
        write_chunk(i, rows0, semw0)
        wait_write(i, rows0, semw0)

        @pl.when(r < N_EMB_CH // 2 - 1)
        def _():
            gather_chunk(i + 2, rows0, semg0)

        wait_gather(rows1, semg1)
        write_chunk(i + 1, rows1, semw1)
        wait_write(i + 1, rows1, semw1)
        return carry

    lax.fori_loop(0, N_EMB_CH // 2, emb_body, 0)

    def nw_drain(c, carry):
        pltpu.make_async_copy(nodew_hbm.at[pl.ds(0, NW_CH)],
                              nwv_v.at[pl.ds(c * NW_CH, NW_CH)], semn).wait()
        return carry

    lax.fori_loop(0, N_NW_CH, nw_drain, 0)
    pltpu.sync_copy(nwv_v, nwv_hbm.at[pl.ds(r0, IDS_PW)])


def _sc_edge_kernel(edgem_hbm, tail_hbm, ewcidx_hbm, ewoidx_hbm,
                    ewv_hbm,
                    ewcidx_v, ewoidx_v, tail_v, ewv_v, sem):
    wid = lax.axis_index("s") * NC + lax.axis_index("c")

    for k in range(4):
        pltpu.sync_copy(
            ewcidx_hbm.at[pl.ds(k * N_IDS + wid * IDS_PW, IDS_PW)],
            ewcidx_v.at[pl.ds(k * IDS_PW, IDS_PW)])
        pltpu.sync_copy(
            ewoidx_hbm.at[pl.ds(k * N_IDS + wid * IDS_PW, IDS_PW)],
            ewoidx_v.at[pl.ds(k * IDS_PW, IDS_PW)])
    pltpu.sync_copy(tail_hbm, tail_v)

    def edge_fire(c, carry):
        idx = ewcidx_v.at[pl.ds(c * EDGE_CH, EDGE_CH)]
        pltpu.async_copy(edgem_hbm.at[idx],
                         ewv_v.at[pl.ds(c * EDGE_CH, EDGE_CH)], sem)
        return carry

    lax.fori_loop(0, N_EDGE_CH, edge_fire, 0)

    def edge_drain(c, carry):
        pltpu.make_async_copy(edgem_hbm.at[pl.ds(0, EDGE_CH)],
                              ewv_v.at[pl.ds(c * EDGE_CH, EDGE_CH)], sem).wait()
        return carry

    lax.fori_loop(0, N_EDGE_CH, edge_drain, 0)

    def fix_body(c, carry):
        e = ewoidx_v[pl.ds(c * 16, 16)]
        v = ewv_v[pl.ds(c * 16, 16)]
        idx_t = jnp.maximum(e - EDGE_MAIN, 0)
        tv = plsc.load_gather(tail_v, [idx_t])
        ewv_v[pl.ds(c * 16, 16)] = jnp.where(e >= EDGE_MAIN, tv, v)
        return carry

    lax.fori_loop(0, EDGE_PW // 16, fix_body, 0)
    for k in range(4):
        pltpu.sync_copy(ewv_v.at[pl.ds(k * IDS_PW, IDS_PW)],
                        ewv_hbm.at[pl.ds(k * N_IDS + wid * IDS_PW, IDS_PW)])


@functools.cache
def _sc_emb():
    return pl.kernel(
        _sc_emb_kernel,
        out_type=[
            jax.ShapeDtypeStruct((N_IDS, EMBED), jnp.float32),
            jax.ShapeDtypeStruct((N_IDS,), jnp.float32),
        ],
        mesh=plsc.VectorSubcoreMesh(core_axis_name="c", subcore_axis_name="s"),
        compiler_params=pltpu.CompilerParams(needs_layout_passes=False),
        scratch_types=[
            pltpu.VMEM((IDS_PW,), jnp.int32),
            pltpu.VMEM((EMB_CH, EMBED), jnp.float32),
            pltpu.VMEM((EMB_CH, EMBED), jnp.float32),
            pltpu.VMEM((IDS_PW,), jnp.float32),
            pltpu.SemaphoreType.DMA,
            pltpu.SemaphoreType.DMA,
            pltpu.SemaphoreType.DMA,
            pltpu.SemaphoreType.DMA,
            pltpu.SemaphoreType.DMA,
        ],
    )


@functools.cache
def _sc_edge():
    return pl.kernel(
        _sc_edge_kernel,
        out_type=jax.ShapeDtypeStruct((N_EDGE,), jnp.float32),
        mesh=plsc.VectorSubcoreMesh(core_axis_name="c", subcore_axis_name="s"),
        compiler_params=pltpu.CompilerParams(needs_layout_passes=False),
        scratch_types=[
            pltpu.VMEM((EDGE_PW,), jnp.int32),
            pltpu.VMEM((EDGE_PW,), jnp.int32),
            pltpu.VMEM((64,), jnp.float32),
            pltpu.VMEM((EDGE_PW,), jnp.float32),
            pltpu.SemaphoreType.DMA,
        ],
    )


BB = 128


def _combine_kernel(g_ref, e0_ref, e1_ref, e2_ref, e3_ref, nw_ref, s_ref):
    G = g_ref[...]
    nw = nw_ref[...]
    z = jnp.zeros((BB, P, EMBED), jnp.float32)
    Gp = jnp.concatenate([z, G, z], axis=1)
    m = None
    for e_ref, o in ((e0_ref, 0), (e1_ref, 1), (e2_ref, 3), (e3_ref, 4)):
        prod = Gp[:, o:o + L, :] * e_ref[...]
        m = prod if m is None else jnp.maximum(m, prod)
    nwe = nw[:, :, None]
    y = (1.0 - nwe) * m + nwe * G
    s_ref[...] = jnp.sum(y, axis=1)


def _head_kernel(s_ref, gamma_ref, beta_ref, fcw_ref, fcb_ref, lab_ref,
                 logits_ref, loss_ref):
    s = s_ref[...]
    mean = jnp.mean(s, axis=0, keepdims=True)
    xc = s - mean
    var = jnp.mean(xc * xc, axis=0, keepdims=True)
    xn = xc * lax.rsqrt(var + 1e-5) * gamma_ref[...] + beta_ref[...]
    lin = lax.dot_general(xn, fcw_ref[...], (((1,), (1,)), ((), ())),
                          preferred_element_type=jnp.float32) + fcb_ref[...]
    m1 = jnp.max(lin, axis=1, keepdims=True)
    lse1 = m1 + jnp.log(jnp.sum(jnp.exp(lin - m1), axis=1, keepdims=True))
    logits = lin - lse1
    m2 = jnp.max(logits, axis=1, keepdims=True)
    lse2 = m2 + jnp.log(jnp.sum(jnp.exp(logits - m2), axis=1, keepdims=True))
    lsm = logits - lse2
    cls = lax.broadcasted_iota(jnp.int32, (B, CLASSES), 1)
    picked = jnp.sum(jnp.where(cls == lab_ref[...], lsm, 0.0), axis=1)
    logits_ref[...] = logits
    loss_ref[...] = (-jnp.mean(picked))[None, None]


@jax.jit
def kernel(input_ids, labels, node_emb, edge_w, node_w, gamma, beta, fcW, fcb):
    X = input_ids.astype(jnp.int32)
    x_flat = X.reshape(-1)
    l_pos = jnp.arange(N_IDS, dtype=jnp.int32) % L
    planes = []
    for o in (-2, -1, 1, 2):
        if o < 0:
            nbk = jnp.concatenate([jnp.zeros((-o,), jnp.int32), x_flat[:o]])
        else:
            nbk = jnp.concatenate([x_flat[o:], jnp.zeros((o,), jnp.int32)])
        valid = (l_pos + o >= 0) & (l_pos + o < L)
        nbk = jnp.where(valid, nbk, 0)
        planes.append(jnp.where(nbk == 0, 0, x_flat * VOCAB + nbk))
    ew_flat = jnp.concatenate(planes)
    ewc_flat = jnp.minimum(ew_flat, EDGE_MAIN - 1)
    edge_main = lax.slice(edge_w, (0, 0), (EDGE_MAIN, 1)).reshape(-1)
    edge_tail = lax.slice(edge_w, (EDGE_MAIN, 0), (EDGE_MAIN + 64, 1)).reshape(-1)

    G, NWV = _sc_emb()(node_emb, node_w.reshape(-1), x_flat)
    EWV = _sc_edge()(edge_main, edge_tail, ewc_flat, ew_flat)

    ewv_planes = [
        lax.slice(EWV, (k * N_IDS,), ((k + 1) * N_IDS,)).reshape(B, L, 1)
        for k in range(4)
    ]
    s = pl.pallas_call(
        _combine_kernel,
        grid=(B // BB,),
        in_specs=[
            pl.BlockSpec((BB, L, EMBED), lambda i: (i, 0, 0)),
            pl.BlockSpec((BB, L, 1), lambda i: (i, 0, 0)),
            pl.BlockSpec((BB, L, 1), lambda i: (i, 0, 0)),
            pl.BlockSpec((BB, L, 1), lambda i: (i, 0, 0)),
            pl.BlockSpec((BB, L, 1), lambda i: (i, 0, 0)),
            pl.BlockSpec((BB, L), lambda i: (i, 0)),
        ],
        out_specs=pl.BlockSpec((BB, EMBED), lambda i: (i, 0)),
        out_shape=jax.ShapeDtypeStruct((B, EMBED), jnp.float32),
    )(G.reshape(B, L, EMBED), *ewv_planes, NWV.reshape(B, L))

    logits, loss2d = pl.pallas_call(
        _head_kernel,
        out_shape=[
            jax.ShapeDtypeStruct((B, CLASSES), jnp.float32),
            jax.ShapeDtypeStruct((1, 1), jnp.float32),
        ],
    )(s, gamma.reshape(1, EMBED), beta.reshape(1, EMBED), fcW,
      fcb.reshape(1, CLASSES), labels.reshape(B, 1).astype(jnp.int32))

    return (loss2d[0, 0], logits)


# speedup vs baseline: 1.5195x; 1.5195x over previous
import functools

import jax
import jax.numpy as jnp
from jax import lax
from jax.experimental import pallas as pl
from jax.experimental.pallas import tpu as pltpu
from jax.experimental.pallas import tpu_sc as plsc

VOCAB = 5000
EMBED = 128
CLASSES = 20
P = 2
B = 1024
L = 50

NC = 2
NS = 16
NWORK = NC * NS

N_IDS = B * L
N_EDGE = B * L * 2 * P

IDS_PW = N_IDS // NWORK
EDGE_PW = N_EDGE // NWORK
ROWS_PW = B // NWORK

EDGE_CH = 128
N_EDGE_CH = EDGE_PW // EDGE_CH
EMB_CH = 80
N_EMB_CH = IDS_PW // EMB_CH
NW_CH = 64
N_NW_CH = IDS_PW // NW_CH
EDGE_MAIN = 24999936
EDGE_MAX = VOCAB * VOCAB - 1


def _sc_emb_kernel(emb_hbm, nodew_hbm, xidx_hbm,
                   g_hbm, nwv_hbm,
                   xidx_v, rows0, rows1, nwv_v,
                   semg0, semg1, semw0, semw1, semn):
    wid = lax.axis_index("s") * NC + lax.axis_index("c")
    r0 = wid * IDS_PW

    pltpu.sync_copy(xidx_hbm.at[pl.ds(r0, IDS_PW)], xidx_v)

    def nw_fire(c, carry):
        idx = xidx_v.at[pl.ds(c * NW_CH, NW_CH)]
        pltpu.async_copy(nodew_hbm.at[idx], nwv_v.at[pl.ds(c * NW_CH, NW_CH)],
                         semn)
        return carry

    lax.fori_loop(0, N_NW_CH, nw_fire, 0)

    def gather_chunk(c, buf, sem):
        idx = xidx_v.at[pl.ds(c * EMB_CH, EMB_CH)]
        pltpu.async_copy(emb_hbm.at[idx], buf, sem)

    def write_chunk(c, buf, sem):
        return pltpu.async_copy(buf, g_hbm.at[pl.ds(r0 + c * EMB_CH, EMB_CH)],
                                sem)

    def wait_gather(buf, sem):
        pltpu.make_async_copy(emb_hbm.at[pl.ds(0, EMB_CH), :], buf, sem).wait()

    def wait_write(c, buf, sem):
        pltpu.make_async_copy(buf, g_hbm.at[pl.ds(r0 + c * EMB_CH, EMB_CH)],
                              sem).wait()

    gather_chunk(0, rows0, semg0)

    def emb_body(r, carry):
        i = 2 * r
        gather_chunk(i + 1, rows1, semg1)
        wait_gather(rows0, semg0)
        write_chunk(i, rows0, semw0)
        wait_write(i, rows0, semw0)

        @pl.when(r < N_EMB_CH // 2 - 1)
        def _():
            gather_chunk(i + 2, rows0, semg0)

        wait_gather(rows1, semg1)
        write_chunk(i + 1, rows1, semw1)
        wait_write(i + 1, rows1, semw1)
        return carry

    lax.fori_loop(0, N_EMB_CH // 2, emb_body, 0)

    def nw_drain(c, carry):
        pltpu.make_async_copy(nodew_hbm.at[pl.ds(0, NW_CH)],
                              nwv_v.at[pl.ds(c * NW_CH, NW_CH)], semn).wait()
        return carry

    lax.fori_loop(0, N_NW_CH, nw_drain, 0)
    pltpu.sync_copy(nwv_v, nwv_hbm.at[pl.ds(r0, IDS_PW)])


def _sc_edge_kernel(edgem_hbm, tail_hbm, ewcidx_hbm, ewoidx_hbm,
                    ewv_hbm,
                    ewcidx_v, ewoidx_v, tail_v, ewv_v, sem):
    wid = lax.axis_index("s") * NC + lax.axis_index("c")

    pltpu.sync_copy(ewcidx_hbm.at[pl.ds(wid * EDGE_PW, EDGE_PW)], ewcidx_v)
    pltpu.sync_copy(ewoidx_hbm.at[pl.ds(wid * EDGE_PW, EDGE_PW)], ewoidx_v)
    pltpu.sync_copy(tail_hbm, tail_v)

    def edge_fire(c, carry):
        idx = ewcidx_v.at[pl.ds(c * EDGE_CH, EDGE_CH)]
        pltpu.async_copy(edgem_hbm.at[idx],
                         ewv_v.at[pl.ds(c * EDGE_CH, EDGE_CH)], sem)
        return carry

    lax.fori_loop(0, N_EDGE_CH, edge_fire, 0)

    def edge_drain(c, carry):
        pltpu.make_async_copy(edgem_hbm.at[pl.ds(0, EDGE_CH)],
                              ewv_v.at[pl.ds(c * EDGE_CH, EDGE_CH)], sem).wait()
        return carry

    lax.fori_loop(0, N_EDGE_CH, edge_drain, 0)

    def fix_body(c, carry):
        e = ewoidx_v[pl.ds(c * 16, 16)]
        v = ewv_v[pl.ds(c * 16, 16)]
        idx_t = jnp.maximum(e - EDGE_MAIN, 0)
        tv = plsc.load_gather(tail_v, [idx_t])
        ewv_v[pl.ds(c * 16, 16)] = jnp.where(e >= EDGE_MAIN, tv, v)
        return carry

    lax.fori_loop(0, EDGE_PW // 16, fix_body, 0)
    pltpu.sync_copy(ewv_v, ewv_hbm.at[pl.ds(wid * EDGE_PW, EDGE_PW)])


@functools.cache
def _sc_emb():
    return pl.kernel(
        _sc_emb_kernel,
        out_type=[
            jax.ShapeDtypeStruct((N_IDS, EMBED), jnp.float32),
            jax.ShapeDtypeStruct((N_IDS,), jnp.float32),
        ],
        mesh=plsc.VectorSubcoreMesh(core_axis_name="c", subcore_axis_name="s"),
        compiler_params=pltpu.CompilerParams(needs_layout_passes=False),
        scratch_types=[
            pltpu.VMEM((IDS_PW,), jnp.int32),
            pltpu.VMEM((EMB_CH, EMBED), jnp.float32),
            pltpu.VMEM((EMB_CH, EMBED), jnp.float32),
            pltpu.VMEM((IDS_PW,), jnp.float32),
            pltpu.SemaphoreType.DMA,
            pltpu.SemaphoreType.DMA,
            pltpu.SemaphoreType.DMA,
            pltpu.SemaphoreType.DMA,
            pltpu.SemaphoreType.DMA,
        ],
    )


@functools.cache
def _sc_edge():
    return pl.kernel(
        _sc_edge_kernel,
        out_type=jax.ShapeDtypeStruct((N_EDGE,), jnp.float32),
        mesh=plsc.VectorSubcoreMesh(core_axis_name="c", subcore_axis_name="s"),
        compiler_params=pltpu.CompilerParams(needs_layout_passes=False),
        scratch_types=[
            pltpu.VMEM((EDGE_PW,), jnp.int32),
            pltpu.VMEM((EDGE_PW,), jnp.int32),
            pltpu.VMEM((64,), jnp.float32),
            pltpu.VMEM((EDGE_PW,), jnp.float32),
            pltpu.SemaphoreType.DMA,
        ],
    )


BB = 128


def _combine_kernel(g_ref, ew_ref, nw_ref, s_ref):
    G = g_ref[...]
    ew = ew_ref[...]
    nw = nw_ref[...]
    z = jnp.zeros((BB, P, EMBED), jnp.float32)
    Gp = jnp.concatenate([z, G, z], axis=1)
    m = None
    for j, o in enumerate((0, 1, 3, 4)):
        prod = Gp[:, o:o + L, :] * ew[:, :, j:j + 1]
        m = prod if m is None else jnp.maximum(m, prod)
    nwe = nw[:, :, None]
    y = (1.0 - nwe) * m + nwe * G
    s_ref[...] = jnp.sum(y, axis=1)


def _head_kernel(s_ref, gamma_ref, beta_ref, fcw_ref, fcb_ref, lab_ref,
                 logits_ref, loss_ref):
    s = s_ref[...]
    mean = jnp.mean(s, axis=0, keepdims=True)
    xc = s - mean
    var = jnp.mean(xc * xc, axis=0, keepdims=True)
    xn = xc * lax.rsqrt(var + 1e-5) * gamma_ref[...] + beta_ref[...]
    lin = lax.dot_general(xn, fcw_ref[...], (((1,), (1,)), ((), ())),
                          preferred_element_type=jnp.float32) + fcb_ref[...]
    m1 = jnp.max(lin, axis=1, keepdims=True)
    lse1 = m1 + jnp.log(jnp.sum(jnp.exp(lin - m1), axis=1, keepdims=True))
    logits = lin - lse1
    m2 = jnp.max(logits, axis=1, keepdims=True)
    lse2 = m2 + jnp.log(jnp.sum(jnp.exp(logits - m2), axis=1, keepdims=True))
    lsm = logits - lse2
    cls = lax.broadcasted_iota(jnp.int32, (B, CLASSES), 1)
    picked = jnp.sum(jnp.where(cls == lab_ref[...], lsm, 0.0), axis=1)
    logits_ref[...] = logits
    loss_ref[...] = (-jnp.mean(picked))[None, None]


@jax.jit
def kernel(input_ids, labels, node_emb, edge_w, node_w, gamma, beta, fcW, fcb):
    X = input_ids.astype(jnp.int32)
    xp = jnp.pad(X, ((0, 0), (P, P)))
    nb = jnp.stack([xp[:, o:o + L] for o in (0, 1, 3, 4)], axis=-1)
    ewi = X[:, :, None] * VOCAB + nb
    ewi = jnp.where(nb == 0, 0, ewi)
    x_flat = X.reshape(-1)
    ew_flat = ewi.reshape(-1)
    ewc_flat = jnp.minimum(ew_flat, EDGE_MAIN - 1)
    edge_main = lax.slice(edge_w, (0, 0), (EDGE_MAIN, 1)).reshape(-1)
    edge_tail = lax.slice(edge_w, (EDGE_MAIN, 0), (EDGE_MAIN + 64, 1)).reshape(-1)

    G, NWV = _sc_emb()(node_emb, node_w.reshape(-1), x_flat)
    EWV = _sc_edge()(edge_main, edge_tail, ewc_flat, ew_flat)

    s = pl.pallas_call(
        _combine_kernel,
        grid=(B // BB,),
        in_specs=[
            pl.BlockSpec((BB, L, EMBED), lambda i: (i, 0, 0)),
            pl.BlockSpec((BB, L, 2 * P), lambda i: (i, 0, 0)),
            pl.BlockSpec((BB, L), lambda i: (i, 0)),
        ],
        out_specs=pl.BlockSpec((BB, EMBED), lambda i: (i, 0)),
        out_shape=jax.ShapeDtypeStruct((B, EMBED), jnp.float32),
    )(G.reshape(B, L, EMBED), EWV.reshape(B, L, 2 * P), NWV.reshape(B, L))

    logits, loss2d = pl.pallas_call(
        _head_kernel,
        out_shape=[
            jax.ShapeDtypeStruct((B, CLASSES), jnp.float32),
            jax.ShapeDtypeStruct((1, 1), jnp.float32),
        ],
    )(s, gamma.reshape(1, EMBED), beta.reshape(1, EMBED), fcW,
      fcb.reshape(1, CLASSES), labels.reshape(B, 1).astype(jnp.int32))

    return (loss2d[0, 0], logits)
